# baseline (speedup 1.0000x reference)
import functools

import jax
import jax.numpy as jnp
from jax.experimental import pallas as pl
from jax.experimental.pallas import tpu as pltpu

BK = 200


def _body(xt_ref, et_ref, o_ref, e_ref, *, nblk):
    i = pl.program_id(0)

    @pl.when(i == 0)
    def _():
        e_ref[...] = et_ref[...].T

    x = xt_ref[...].astype(jnp.float32)
    e_slab = e_ref[pl.ds(i * BK, BK), :]
    prod_t = jax.lax.dot_general(
        e_slab, x, (((0,), (0,)), ((), ())),
        preferred_element_type=jnp.float32,
    )
    shifted = prod_t[:, 1:]

    @pl.when(i == 0)
    def _():
        o_ref[...] = shifted

    @pl.when(i > 0)
    def _():
        o_ref[...] += shifted


def kernel(inputs, embeddings):
    M, K = inputs.shape
    _, N = embeddings.shape
    xt = inputs.T
    et = embeddings.T
    nblk = K // BK
    out_t = pl.pallas_call(
        functools.partial(_body, nblk=nblk),
        grid=(nblk,),
        in_specs=[
            pl.BlockSpec((BK, M), lambda i: (i, 0)),
            pl.BlockSpec((N, K), lambda i: (0, 0)),
        ],
        out_specs=pl.BlockSpec((N, M - 1), lambda i: (0, 0)),
        out_shape=jax.ShapeDtypeStruct((N, M - 1), jnp.float32),
        scratch_shapes=[
            pltpu.VMEM((K, N), jnp.float32),
        ],
    )(xt, et)
    return out_t.T
